# Initial kernel scaffold; baseline (speedup 1.0000x reference)
import jax
import jax.numpy as jnp
from jax import lax
from jax.experimental import pallas as pl
from jax.experimental.pallas import tpu as pltpu
from jax.experimental.pallas import tpu_sc as plsc

_VOCAB = 1000
_N = 16384 * 200
_NW = 32
_PER_W = _N // _NW
_CHUNK = 12800
_NCHUNK = _PER_W // _CHUNK
_VPC = _CHUNK // 16


def _gather_kernel(index_hbm, table_hbm, out_hbm, table_v, idx_v, out_v):
    wid = lax.axis_index("s") * 2 + lax.axis_index("c")
    base = wid * _PER_W
    pltpu.sync_copy(table_hbm, table_v)

    def chunk_body(c, carry):
        off = pl.multiple_of(base + c * _CHUNK, _CHUNK)
        pltpu.sync_copy(index_hbm.at[pl.ds(off, _CHUNK)], idx_v)

        def vec_body(i, carry2):
            idx = idx_v[pl.ds(i * 16, 16)]
            out_v[pl.ds(i * 16, 16)] = plsc.load_gather(table_v, [idx])
            return carry2

        lax.fori_loop(0, _VPC, vec_body, 0, unroll=4)
        pltpu.sync_copy(out_v, out_hbm.at[pl.ds(off, _CHUNK)])
        return carry

    lax.fori_loop(0, _NCHUNK, chunk_body, 0)


@jax.jit
def kernel(index, table):
    flat = index.reshape(_N)
    mesh = plsc.VectorSubcoreMesh(core_axis_name="c", subcore_axis_name="s")
    run = pl.kernel(
        _gather_kernel,
        out_type=jax.ShapeDtypeStruct((_N,), jnp.float32),
        mesh=mesh,
        scratch_types=[
            pltpu.VMEM((_VOCAB,), jnp.float32),
            pltpu.VMEM((_CHUNK,), jnp.int32),
            pltpu.VMEM((_CHUNK,), jnp.float32),
        ],
    )
    out = run(flat, table)
    return out.reshape(index.shape)


# speedup vs baseline: 154.3932x; 154.3932x over previous
import jax
import jax.numpy as jnp
from jax import lax
from jax.experimental import pallas as pl
from jax.experimental.pallas import tpu as pltpu
from jax.experimental.pallas import tpu_sc as plsc

_VOCAB = 1000
_N = 16384 * 200
_NW = 32
_PER_W = _N // _NW
_CHUNK = 12800
_NCHUNK = _PER_W // _CHUNK
_VPC = _CHUNK // 16


def _gather_kernel(index_hbm, table_hbm, out_hbm, table_v, idx_v, out_v):
    wid = lax.axis_index("s") * 2 + lax.axis_index("c")
    base = wid * _PER_W
    pltpu.sync_copy(table_hbm, table_v)

    def chunk_body(c, carry):
        off = pl.multiple_of(base + c * _CHUNK, _CHUNK)
        pltpu.sync_copy(index_hbm.at[pl.ds(off, _CHUNK)], idx_v)

        def vec_body(i, carry2):
            idx = idx_v[pl.ds(i * 16, 16)]
            out_v[pl.ds(i * 16, 16)] = plsc.load_gather(table_v, [idx])
            return carry2

        lax.fori_loop(0, _VPC, vec_body, 0, unroll=4)
        pltpu.sync_copy(out_v, out_hbm.at[pl.ds(off, _CHUNK)])
        return carry

    lax.fori_loop(0, _NCHUNK, chunk_body, 0)


@jax.jit
def kernel(index, table):
    flat = index.reshape(_N)
    mesh = plsc.VectorSubcoreMesh(core_axis_name="c", subcore_axis_name="s")
    run = pl.kernel(
        _gather_kernel,
        out_type=jax.ShapeDtypeStruct((_N,), jnp.float32),
        mesh=mesh,
        scratch_types=[
            pltpu.VMEM((_VOCAB,), jnp.float32),
            pltpu.VMEM((_CHUNK,), jnp.int32),
            pltpu.VMEM((_CHUNK,), jnp.float32),
        ],
        compiler_params=pltpu.CompilerParams(
            needs_layout_passes=False,
            use_tc_tiling_on_sc=False,
        ),
    )
    out = run(flat, table)
    return out.reshape(index.shape)


# speedup vs baseline: 233.6270x; 1.5132x over previous
import jax
import jax.numpy as jnp
from jax import lax
from jax.experimental import pallas as pl
from jax.experimental.pallas import tpu as pltpu
from jax.experimental.pallas import tpu_sc as plsc

_VOCAB = 1000
_N = 16384 * 200
_NW = 32
_PER_W = _N // _NW
_CHUNK = 12800
_NCHUNK = _PER_W // _CHUNK
_U = 8
_STEPS = _CHUNK // (16 * _U)


def _gather_kernel(index_hbm, table_hbm, out_hbm,
                   table_v, idx_v, out_v, isem0, isem1, osem0, osem1):
    wid = lax.axis_index("s") * 2 + lax.axis_index("c")
    base = wid * _PER_W
    pltpu.sync_copy(table_hbm, table_v)

    isems = (isem0, isem1)
    osems = (osem0, osem1)

    def start_in(c, b):
        off = pl.multiple_of(base + c * _CHUNK, _CHUNK)
        return pltpu.async_copy(
            index_hbm.at[pl.ds(off, _CHUNK)], idx_v.at[b], isems[b])

    def start_out(c, b):
        off = pl.multiple_of(base + c * _CHUNK, _CHUNK)
        return pltpu.async_copy(
            out_v.at[b], out_hbm.at[pl.ds(off, _CHUNK)], osems[b])

    def compute(b):
        def step(i, carry):
            s = i * (16 * _U)
            idxs = [idx_v[b, pl.ds(s + k * 16, 16)] for k in range(_U)]
            vals = [plsc.load_gather(table_v, [ix]) for ix in idxs]
            for k in range(_U):
                out_v[b, pl.ds(s + k * 16, 16)] = vals[k]
            return carry

        lax.fori_loop(0, _STEPS, step, 0)

    in_handles = {}
    out_handles = {}
    in_handles[0] = start_in(0, 0)
    in_handles[1] = start_in(1, 1)
    for c in range(_NCHUNK):
        b = c % 2
        in_handles[c].wait()
        if c >= 2:
            out_handles[c - 2].wait()
        compute(b)
        out_handles[c] = start_out(c, b)
        if c + 2 < _NCHUNK:
            in_handles[c + 2] = start_in(c + 2, b)
    out_handles[_NCHUNK - 2].wait()
    out_handles[_NCHUNK - 1].wait()


@jax.jit
def kernel(index, table):
    flat = index.reshape(_N)
    mesh = plsc.VectorSubcoreMesh(core_axis_name="c", subcore_axis_name="s")
    run = pl.kernel(
        _gather_kernel,
        out_type=jax.ShapeDtypeStruct((_N,), jnp.float32),
        mesh=mesh,
        scratch_types=[
            pltpu.VMEM((_VOCAB,), jnp.float32),
            pltpu.VMEM((2, _CHUNK), jnp.int32),
            pltpu.VMEM((2, _CHUNK), jnp.float32),
            pltpu.SemaphoreType.DMA,
            pltpu.SemaphoreType.DMA,
            pltpu.SemaphoreType.DMA,
            pltpu.SemaphoreType.DMA,
        ],
        compiler_params=pltpu.CompilerParams(
            needs_layout_passes=False,
            use_tc_tiling_on_sc=False,
        ),
    )
    out = run(flat, table)
    return out.reshape(index.shape)


# speedup vs baseline: 416.4508x; 1.7825x over previous
import jax
import jax.numpy as jnp
from jax import lax
from jax.experimental import pallas as pl
from jax.experimental.pallas import tpu as pltpu
from jax.experimental.pallas import tpu_sc as plsc

_VOCAB = 1000
_ROWS = 16384
_COLS = 200
_NW = 32
_ROWS_W = _ROWS // _NW
_RCHUNK = 64
_NCHUNK = _ROWS_W // _RCHUNK
_OFFS = tuple(range(0, 192, 16)) + (184,)


def _gather_kernel(index_hbm, table_hbm, out_hbm,
                   table_v, idx_v, out_v, isem0, isem1, osem0, osem1):
    wid = lax.axis_index("s") * 2 + lax.axis_index("c")
    row0 = wid * _ROWS_W
    pltpu.sync_copy(table_hbm, table_v)

    def start_in(c, b):
        r = pl.multiple_of(row0 + c * _RCHUNK, _RCHUNK)
        return pltpu.async_copy(
            index_hbm.at[pl.ds(r, _RCHUNK)], idx_v.at[b],
            (isem0, isem1)[b])

    def start_out(c, b):
        r = pl.multiple_of(row0 + c * _RCHUNK, _RCHUNK)
        return pltpu.async_copy(
            out_v.at[b], out_hbm.at[pl.ds(r, _RCHUNK)],
            (osem0, osem1)[b])

    def compute(b):
        def step(i, carry):
            for rr in range(2):
                r = i * 2 + rr
                idxs = [idx_v[b, r, pl.ds(o, 16)] for o in _OFFS]
                vals = [plsc.load_gather(table_v, [ix]) for ix in idxs]
                for o, v in zip(_OFFS, vals):
                    out_v[b, r, pl.ds(o, 16)] = v
            return carry

        lax.fori_loop(0, _RCHUNK // 2, step, 0)

    in_handles = {}
    out_handles = {}
    in_handles[0] = start_in(0, 0)
    in_handles[1] = start_in(1, 1)
    for c in range(_NCHUNK):
        b = c % 2
        in_handles[c].wait()
        if c >= 2:
            out_handles[c - 2].wait()
        compute(b)
        out_handles[c] = start_out(c, b)
        if c + 2 < _NCHUNK:
            in_handles[c + 2] = start_in(c + 2, b)
    out_handles[_NCHUNK - 2].wait()
    out_handles[_NCHUNK - 1].wait()


@jax.jit
def kernel(index, table):
    mesh = plsc.VectorSubcoreMesh(core_axis_name="c", subcore_axis_name="s")
    run = pl.kernel(
        _gather_kernel,
        out_type=jax.ShapeDtypeStruct((_ROWS, _COLS), jnp.float32),
        mesh=mesh,
        scratch_types=[
            pltpu.VMEM((_VOCAB,), jnp.float32),
            pltpu.VMEM((2, _RCHUNK, _COLS), jnp.int32),
            pltpu.VMEM((2, _RCHUNK, _COLS), jnp.float32),
            pltpu.SemaphoreType.DMA,
            pltpu.SemaphoreType.DMA,
            pltpu.SemaphoreType.DMA,
            pltpu.SemaphoreType.DMA,
        ],
        compiler_params=pltpu.CompilerParams(
            needs_layout_passes=False,
            use_tc_tiling_on_sc=True,
        ),
    )
    return run(index, table)


# speedup vs baseline: 732.2040x; 1.7582x over previous
import jax
import jax.numpy as jnp
from jax import lax
from jax.experimental import pallas as pl
from jax.experimental.pallas import tpu as pltpu
from jax.experimental.pallas import tpu_sc as plsc

_VOCAB = 1000
_RT = 200
_CT = 16384
_NW = 32
_COLS_W = _CT // _NW
_RCHUNK = 40
_NCHUNK = _RT // _RCHUNK
_NSLICE = _COLS_W // 16


def _gather_kernel(index_hbm, table_hbm, out_hbm,
                   table_v, idx_v, out_v, isem0, isem1, osem0, osem1):
    wid = lax.axis_index("s") * 2 + lax.axis_index("c")
    col0 = wid * _COLS_W
    pltpu.sync_copy(table_hbm, table_v)

    def start_in(c, b):
        return pltpu.async_copy(
            index_hbm.at[pl.ds(c * _RCHUNK, _RCHUNK), pl.ds(col0, _COLS_W)],
            idx_v.at[b], (isem0, isem1)[b])

    def start_out(c, b):
        return pltpu.async_copy(
            out_v.at[b],
            out_hbm.at[pl.ds(c * _RCHUNK, _RCHUNK), pl.ds(col0, _COLS_W)],
            (osem0, osem1)[b])

    def compute(b):
        def step(r, carry):
            idxs = [idx_v[b, r, pl.ds(o * 16, 16)] for o in range(_NSLICE)]
            vals = [plsc.load_gather(table_v, [ix]) for ix in idxs]
            for o, v in enumerate(vals):
                out_v[b, r, pl.ds(o * 16, 16)] = v
            return carry

        lax.fori_loop(0, _RCHUNK, step, 0)

    in_handles = {}
    out_handles = {}
    in_handles[0] = start_in(0, 0)
    in_handles[1] = start_in(1, 1)
    for c in range(_NCHUNK):
        b = c % 2
        in_handles[c].wait()
        if c >= 2:
            out_handles[c - 2].wait()
        compute(b)
        out_handles[c] = start_out(c, b)
        if c + 2 < _NCHUNK:
            in_handles[c + 2] = start_in(c + 2, b)
    out_handles[_NCHUNK - 2].wait()
    out_handles[_NCHUNK - 1].wait()


@jax.jit
def kernel(index, table):
    mesh = plsc.VectorSubcoreMesh(core_axis_name="c", subcore_axis_name="s")
    run = pl.kernel(
        _gather_kernel,
        out_type=jax.ShapeDtypeStruct((_RT, _CT), jnp.float32),
        mesh=mesh,
        scratch_types=[
            pltpu.VMEM((_VOCAB,), jnp.float32),
            pltpu.VMEM((2, _RCHUNK, _COLS_W), jnp.int32),
            pltpu.VMEM((2, _RCHUNK, _COLS_W), jnp.float32),
            pltpu.SemaphoreType.DMA,
            pltpu.SemaphoreType.DMA,
            pltpu.SemaphoreType.DMA,
            pltpu.SemaphoreType.DMA,
        ],
        compiler_params=pltpu.CompilerParams(
            needs_layout_passes=False,
            use_tc_tiling_on_sc=True,
        ),
    )
    out_t = run(index.T, table)
    return out_t.T
